# baseline (speedup 1.0000x reference)
import functools

import jax
import jax.numpy as jnp
from jax import lax
from jax.experimental import pallas as pl
from jax.experimental.pallas import tpu as pltpu
from jax.experimental.pallas import tpu_sc as plsc

VOCAB = 50000
D = 4096
B = 4 * 2048

_INFO = plsc.get_sparse_core_info()
NW = _INFO.num_cores * _INFO.num_subcores
B_PER_W = B // NW
CHUNK = 4
NBUF = 7
LA = 5
N_STEPS = B_PER_W // CHUNK
_MAIN = (N_STEPS // NBUF) * NBUF


def _sc_gather(ids_flat, table):
    mesh = plsc.VectorSubcoreMesh(core_axis_name="c", subcore_axis_name="s")

    @functools.partial(
        pl.kernel,
        out_type=jax.ShapeDtypeStruct((B, D), jnp.float32),
        mesh=mesh,
        scratch_types=(
            [pltpu.VMEM((N_STEPS, CHUNK), jnp.int32)]
            + [pltpu.VMEM((CHUNK, D), jnp.float32) for _ in range(NBUF)]
            + [pltpu.SemaphoreType.DMA for _ in range(2 * NBUF)]
        ),
    )
    def body(ids_hbm, table_hbm, out_hbm, idx_v, *scratch):
        bufs = scratch[:NBUF]
        sem_g = scratch[NBUF : 2 * NBUF]
        sem_w = scratch[2 * NBUF :]
        wid = lax.axis_index("s") * _INFO.num_cores + lax.axis_index("c")
        base = wid * B_PER_W
        pltpu.sync_copy(ids_hbm.at[pl.ds(wid * N_STEPS, N_STEPS)], idx_v)

        def start_gather(c, b):
            pltpu.make_async_copy(
                table_hbm.at[idx_v.at[c]], bufs[b], sem_g[b]
            ).start()

        def wait_gather(b):
            pltpu.make_async_copy(
                table_hbm.at[pl.ds(0, CHUNK)], bufs[b], sem_g[b]
            ).wait()

        def start_wb(c, b):
            pltpu.make_async_copy(
                bufs[b], out_hbm.at[pl.ds(base + c * CHUNK, CHUNK)], sem_w[b]
            ).start()

        def wait_wb(b):
            pltpu.make_async_copy(
                bufs[b], out_hbm.at[pl.ds(base, CHUNK)], sem_w[b]
            ).wait()

        def step(c, j):
            b2 = (j + LA) % NBUF
            wait_gather(j)
            start_wb(c, j)
            c2 = c + LA

            if isinstance(c, int):
                if c2 < N_STEPS:
                    if c >= NBUF - LA:
                        wait_wb(b2)
                    start_gather(c2, b2)
            else:

                @pl.when(c2 < N_STEPS)
                def _():
                    @pl.when(c >= NBUF - LA)
                    def _():
                        wait_wb(b2)

                    start_gather(c2, b2)

        for j in range(LA):
            start_gather(j, j)

        def outer(g, carry):
            for j in range(NBUF):
                step(g * NBUF + j, j)
            return carry

        lax.fori_loop(0, N_STEPS // NBUF, outer, 0)
        for c in range(_MAIN, N_STEPS):
            step(c, c % NBUF)
        for b in range(NBUF):
            wait_wb(b)

    return body(ids_flat, table)


def kernel(input_ids, embedding):
    ids_flat = input_ids.reshape(B // CHUNK, CHUNK).astype(jnp.int32)
    out = _sc_gather(ids_flat, embedding)
    return out.reshape(input_ids.shape[0], input_ids.shape[1], D)


# speedup vs baseline: 1.0044x; 1.0044x over previous
import functools

import jax
import jax.numpy as jnp
from jax import lax
from jax.experimental import pallas as pl
from jax.experimental.pallas import tpu as pltpu
from jax.experimental.pallas import tpu_sc as plsc

VOCAB = 50000
D = 4096
B = 4 * 2048

_INFO = plsc.get_sparse_core_info()
NW = _INFO.num_cores * _INFO.num_subcores
B_PER_W = B // NW
CHUNK = 4
NBUF = 7
LA = 4
N_STEPS = B_PER_W // CHUNK
_MAIN = (N_STEPS // NBUF) * NBUF


def _sc_gather(ids_flat, table):
    mesh = plsc.VectorSubcoreMesh(core_axis_name="c", subcore_axis_name="s")

    @functools.partial(
        pl.kernel,
        out_type=jax.ShapeDtypeStruct((B, D), jnp.float32),
        mesh=mesh,
        scratch_types=(
            [pltpu.VMEM((N_STEPS, CHUNK), jnp.int32)]
            + [pltpu.VMEM((CHUNK, D), jnp.float32) for _ in range(NBUF)]
            + [pltpu.SemaphoreType.DMA for _ in range(2 * NBUF)]
        ),
    )
    def body(ids_hbm, table_hbm, out_hbm, idx_v, *scratch):
        bufs = scratch[:NBUF]
        sem_g = scratch[NBUF : 2 * NBUF]
        sem_w = scratch[2 * NBUF :]
        wid = lax.axis_index("s") * _INFO.num_cores + lax.axis_index("c")
        base = wid * B_PER_W
        pltpu.sync_copy(ids_hbm.at[pl.ds(wid * N_STEPS, N_STEPS)], idx_v)

        def start_gather(c, b):
            pltpu.make_async_copy(
                table_hbm.at[idx_v.at[c]], bufs[b], sem_g[b]
            ).start()

        def wait_gather(b):
            pltpu.make_async_copy(
                table_hbm.at[pl.ds(0, CHUNK)], bufs[b], sem_g[b]
            ).wait()

        def start_wb(c, b):
            pltpu.make_async_copy(
                bufs[b], out_hbm.at[pl.ds(base + c * CHUNK, CHUNK)], sem_w[b]
            ).start()

        def wait_wb(b):
            pltpu.make_async_copy(
                bufs[b], out_hbm.at[pl.ds(base, CHUNK)], sem_w[b]
            ).wait()

        def step(c, j):
            b2 = (j + LA) % NBUF
            wait_gather(j)
            start_wb(c, j)
            c2 = c + LA

            if isinstance(c, int):
                if c2 < N_STEPS:
                    if c >= NBUF - LA:
                        wait_wb(b2)
                    start_gather(c2, b2)
            else:

                @pl.when(c2 < N_STEPS)
                def _():
                    @pl.when(c >= NBUF - LA)
                    def _():
                        wait_wb(b2)

                    start_gather(c2, b2)

        for j in range(LA):
            start_gather(j, j)

        def outer(g, carry):
            for j in range(NBUF):
                step(g * NBUF + j, j)
            return carry

        lax.fori_loop(0, N_STEPS // NBUF, outer, 0)
        for c in range(_MAIN, N_STEPS):
            step(c, c % NBUF)
        for b in range(NBUF):
            wait_wb(b)

    return body(ids_flat, table)


def kernel(input_ids, embedding):
    ids_flat = input_ids.reshape(B // CHUNK, CHUNK).astype(jnp.int32)
    out = _sc_gather(ids_flat, embedding)
    return out.reshape(input_ids.shape[0], input_ids.shape[1], D)
